# baseline (speedup 1.0000x reference)
import functools

import jax
import jax.numpy as jnp
from jax import lax
from jax.experimental import pallas as pl
from jax.experimental.pallas import tpu as pltpu
from jax.experimental.pallas import tpu_sc as plsc

_NC = 2
_NS = 16
_NW = _NC * _NS
_LANES = 16

_GATHER = 128
_BLKS = 4
_CHUNK = _GATHER * _BLKS


def _rsqrt16(x):
    i = plsc.bitcast(x, jnp.int32)
    i = jnp.int32(0x5F3759DF) - lax.shift_right_logical(i, 1)
    y = plsc.bitcast(i, jnp.float32)
    nh = x * jnp.float32(-0.5)
    for _ in range(3):
        y = y * (jnp.float32(1.5) + nh * y * y)
    return y


def _make_sc_kernel(n_rows, d):
    assert d == 4 * _LANES
    per_w = n_rows // _NW
    assert per_w * _NW == n_rows
    n_chunks = per_w // _CHUNK
    assert n_chunks * _CHUNK == per_w and n_chunks % 2 == 0
    idx_rows = per_w // _GATHER
    mesh = plsc.VectorSubcoreMesh(
        core_axis_name="c", subcore_axis_name="s",
        num_cores=_NC, num_subcores=_NS)

    @functools.partial(
        pl.kernel,
        out_type=jax.ShapeDtypeStruct((n_rows, d), jnp.float32),
        mesh=mesh,
        compiler_params=pltpu.CompilerParams(
            needs_layout_passes=False, use_tc_tiling_on_sc=False),
        scratch_types=[
            pltpu.VMEM((idx_rows, _GATHER), jnp.int32),
            pltpu.VMEM((_CHUNK, d), jnp.float32),
            pltpu.VMEM((_CHUNK, d), jnp.float32),
            pltpu.VMEM((d,), jnp.float32),
            pltpu.VMEM((d,), jnp.float32),
            pltpu.SemaphoreType.DMA,
            pltpu.SemaphoreType.DMA,
            pltpu.SemaphoreType.DMA,
            pltpu.SemaphoreType.DMA,
        ],
    )
    def sc_kernel(x_hbm, table_hbm, gamma_hbm, beta_hbm, out_hbm,
                  idx_v, rows0_v, rows1_v,
                  g_v, b_v, sg0, sg1, sw0, sw1):
        wid = lax.axis_index("s") * _NC + lax.axis_index("c")
        rows = (rows0_v, rows1_v)
        sg = (sg0, sg1)
        sw = (sw0, sw1)
        base_row = wid * per_w

        pltpu.sync_copy(x_hbm.at[pl.ds(wid * idx_rows, idx_rows)], idx_v)
        pltpu.sync_copy(gamma_hbm, g_v)
        pltpu.sync_copy(beta_hbm, b_v)

        def fire_gather(cc, bi):
            for j in range(_BLKS):
                pltpu.async_copy(
                    table_hbm.at[idx_v.at[cc * _BLKS + j]],
                    rows[bi].at[pl.ds(j * _GATHER, _GATHER)], sg[bi])

        def drain_gather(cc, bi):
            for j in range(_BLKS):
                pltpu.make_async_copy(
                    table_hbm.at[idx_v.at[cc * _BLKS + j]],
                    rows[bi].at[pl.ds(j * _GATHER, _GATHER)], sg[bi]).wait()

        def wb_descr(cc, bi):
            return pltpu.make_async_copy(
                rows[bi], out_hbm.at[pl.ds(base_row + cc * _CHUNK, _CHUNK)],
                sw[bi])

        def compute(bi):
            rv = rows[bi]
            lane = lax.iota(jnp.int32, _LANES)
            gv = [g_v[pl.ds(k * _LANES, _LANES)] for k in range(4)]
            bv = [b_v[pl.ds(k * _LANES, _LANES)] for k in range(4)]

            @pl.loop(0, _CHUNK // _LANES)
            def _group(gi):
                ridx = gi * _LANES + lane
                m = jnp.zeros((_LANES,), jnp.float32)
                q = jnp.zeros((_LANES,), jnp.float32)
                for dd in range(d):
                    cidx = jnp.full((_LANES,), dd, jnp.int32)
                    col = plsc.load_gather(rv, [ridx, cidx])
                    m = m + col
                    q = q + col * col
                mean = m * jnp.float32(1.0 / 64.0)
                var = q * jnp.float32(1.0 / 64.0) - mean * mean
                rstd = _rsqrt16(var + jnp.float32(1e-5))
                for dd in range(d):
                    cidx = jnp.full((_LANES,), dd, jnp.int32)
                    col = plsc.load_gather(rv, [ridx, cidx])
                    a = rstd * gv[dd // _LANES][dd % _LANES]
                    cc2 = bv[dd // _LANES][dd % _LANES] - mean * a
                    plsc.store_scatter(rv, [ridx, cidx], col * a + cc2)

        fire_gather(0, 0)

        @pl.loop(0, n_chunks, step=2)
        def _chunks(c):
            for bi in range(2):
                cc = c + bi

                @pl.when(cc + 1 < n_chunks)
                def _fire_next():
                    @pl.when(cc >= 1)
                    def _wb_done():
                        wb_descr(cc - 1, bi ^ 1).wait()
                    fire_gather(cc + 1, bi ^ 1)

                drain_gather(cc, bi)
                compute(bi)
                wb_descr(cc, bi).start()

        wb_descr(n_chunks - 2, 0).wait()
        wb_descr(n_chunks - 1, 1).wait()

    return sc_kernel


def kernel(x, table, gamma, beta):
    bsz, seq = x.shape
    d = table.shape[1]
    n = bsz * seq
    x2 = x.reshape(n // _GATHER, _GATHER).astype(jnp.int32)
    out = _make_sc_kernel(n, d)(x2, table, gamma, beta)
    return out.reshape(bsz, seq, d)


# speedup vs baseline: 2.5187x; 2.5187x over previous
import functools

import jax
import jax.numpy as jnp
from jax import lax
from jax.experimental import pallas as pl
from jax.experimental.pallas import tpu as pltpu
from jax.experimental.pallas import tpu_sc as plsc

_NC = 2
_NS = 16
_NW = _NC * _NS
_LANES = 16

_GATHER = 128
_BLKS = 4
_CHUNK = _GATHER * _BLKS


def _rsqrt16(x):
    i = plsc.bitcast(x, jnp.int32)
    i = jnp.int32(0x5F3759DF) - lax.shift_right_logical(i, 1)
    y = plsc.bitcast(i, jnp.float32)
    nh = x * jnp.float32(-0.5)
    for _ in range(3):
        y = y * (jnp.float32(1.5) + nh * y * y)
    return y


def _make_sc_kernel(n_rows, d):
    assert d == 4 * _LANES
    per_w = n_rows // _NW
    assert per_w * _NW == n_rows
    n_chunks = per_w // _CHUNK
    assert n_chunks * _CHUNK == per_w and n_chunks % 2 == 0
    idx_rows = per_w // _GATHER
    mesh = plsc.VectorSubcoreMesh(
        core_axis_name="c", subcore_axis_name="s",
        num_cores=_NC, num_subcores=_NS)

    @functools.partial(
        pl.kernel,
        out_type=jax.ShapeDtypeStruct((n_rows, d), jnp.float32),
        mesh=mesh,
        compiler_params=pltpu.CompilerParams(
            needs_layout_passes=False, use_tc_tiling_on_sc=False),
        scratch_types=[
            pltpu.VMEM((idx_rows, _GATHER), jnp.int32),
            pltpu.VMEM((_CHUNK, d), jnp.float32),
            pltpu.VMEM((_CHUNK, d), jnp.float32),
            pltpu.VMEM((d,), jnp.float32),
            pltpu.VMEM((d,), jnp.float32),
            pltpu.SemaphoreType.DMA,
            pltpu.SemaphoreType.DMA,
            pltpu.SemaphoreType.DMA,
            pltpu.SemaphoreType.DMA,
        ],
    )
    def sc_kernel(x_hbm, table_hbm, gamma_hbm, beta_hbm, out_hbm,
                  idx_v, rows0_v, rows1_v,
                  g_v, b_v, sg0, sg1, sw0, sw1):
        wid = lax.axis_index("s") * _NC + lax.axis_index("c")
        rows = (rows0_v, rows1_v)
        sg = (sg0, sg1)
        sw = (sw0, sw1)
        base_row = wid * per_w

        pltpu.sync_copy(x_hbm.at[pl.ds(wid * idx_rows, idx_rows)], idx_v)
        pltpu.sync_copy(gamma_hbm, g_v)
        pltpu.sync_copy(beta_hbm, b_v)

        def fire_gather(cc, bi):
            for j in range(_BLKS):
                pltpu.async_copy(
                    table_hbm.at[idx_v.at[cc * _BLKS + j]],
                    rows[bi].at[pl.ds(j * _GATHER, _GATHER)], sg[bi])

        def drain_gather(cc, bi):
            for j in range(_BLKS):
                pltpu.make_async_copy(
                    table_hbm.at[idx_v.at[cc * _BLKS + j]],
                    rows[bi].at[pl.ds(j * _GATHER, _GATHER)], sg[bi]).wait()

        def wb_descr(cc, bi):
            return pltpu.make_async_copy(
                rows[bi], out_hbm.at[pl.ds(base_row + cc * _CHUNK, _CHUNK)],
                sw[bi])

        def compute(bi):
            rv = rows[bi]
            gv = [g_v[pl.ds(k * _LANES, _LANES)] for k in range(4)]
            bv = [b_v[pl.ds(k * _LANES, _LANES)] for k in range(4)]

            @pl.loop(0, _CHUNK, unroll=4)
            def _row(r):
                v = [rv[r, pl.ds(k * _LANES, _LANES)] for k in range(4)]
                s = (v[0] + v[1]) + (v[2] + v[3])
                q = (v[0] * v[0] + v[1] * v[1]) + (v[2] * v[2] + v[3] * v[3])
                mean = jnp.sum(s) * jnp.float32(1.0 / 64.0)
                ex2 = jnp.sum(q) * jnp.float32(1.0 / 64.0)
                xe = ex2 - mean * mean + jnp.float32(1e-5)
                i = lax.bitcast_convert_type(xe, jnp.int32)
                i = jnp.int32(0x5F3759DF) - lax.shift_right_logical(i, 1)
                y = lax.bitcast_convert_type(i, jnp.float32)
                nh = xe * jnp.float32(-0.5)
                for _ in range(3):
                    y = y * (jnp.float32(1.5) + nh * y * y)
                rs = jnp.full((_LANES,), y, jnp.float32)
                tm = jnp.full((_LANES,), mean * y, jnp.float32)
                for k in range(4):
                    a = rs * gv[k]
                    cc2 = bv[k] - tm * gv[k]
                    rv[r, pl.ds(k * _LANES, _LANES)] = v[k] * a + cc2

        fire_gather(0, 0)

        @pl.loop(0, n_chunks, step=2)
        def _chunks(c):
            for bi in range(2):
                cc = c + bi

                @pl.when(cc + 1 < n_chunks)
                def _fire_next():
                    @pl.when(cc >= 1)
                    def _wb_done():
                        wb_descr(cc - 1, bi ^ 1).wait()
                    fire_gather(cc + 1, bi ^ 1)

                drain_gather(cc, bi)
                compute(bi)
                wb_descr(cc, bi).start()

        wb_descr(n_chunks - 2, 0).wait()
        wb_descr(n_chunks - 1, 1).wait()

    return sc_kernel


def kernel(x, table, gamma, beta):
    bsz, seq = x.shape
    d = table.shape[1]
    n = bsz * seq
    x2 = x.reshape(n // _GATHER, _GATHER).astype(jnp.int32)
    out = _make_sc_kernel(n, d)(x2, table, gamma, beta)
    return out.reshape(bsz, seq, d)
